# baseline (speedup 1.0000x reference)
import functools

import jax
import jax.numpy as jnp
from jax import lax
from jax.experimental import pallas as pl
from jax.experimental.pallas import tpu as pltpu
from jax.experimental.pallas import tpu_sc as plsc

H = 768
F = 4 * H
E = 8
TM = 128
T = 2048
NT = T // TM + E
TPAD = NT * TM
NTE = 32


def _router_body(x_ref, rw_ref, rb_ref, pos_ref, po_ref, nt_ref):
    x = x_ref[...]
    rw = rw_ref[...]
    rb = rb_ref[...]
    logits = jax.lax.dot_general(
        x, rw, (((1,), (1,)), ((), ())),
        preferred_element_type=jnp.float32) + rb
    e_iota = jax.lax.broadcasted_iota(jnp.int32, (T, E), 1)
    m = jnp.max(logits, axis=1, keepdims=True)
    eid = jnp.min(jnp.where(logits == m, e_iota, E), axis=1, keepdims=True)
    onehot = (e_iota == eid).astype(jnp.float32)
    r_i = jax.lax.broadcasted_iota(jnp.int32, (T, T), 0)
    c_i = jax.lax.broadcasted_iota(jnp.int32, (T, T), 1)
    lt = (c_i < r_i).astype(jnp.float32)
    rank = jax.lax.dot_general(
        lt, onehot, (((1,), (0,)), ((), ())),
        preferred_element_type=jnp.float32)
    ones_col = jnp.full((T, 1), 1.0, dtype=jnp.float32)
    cntf = jax.lax.dot_general(
        onehot, ones_col, (((0,), (0,)), ((), ())),
        preferred_element_type=jnp.float32)
    ntiles = (cntf.astype(jnp.int32) + (TM - 1)) // TM
    pcf = (ntiles * TM).astype(jnp.float32)
    ei = jax.lax.broadcasted_iota(jnp.int32, (E, E), 0)
    ej = jax.lax.broadcasted_iota(jnp.int32, (E, E), 1)
    ltE = (ej < ei).astype(jnp.float32)
    pof = jax.lax.dot_general(
        ltE, pcf, (((1,), (0,)), ((), ())),
        preferred_element_type=jnp.float32)
    pos_sel = jax.lax.dot_general(
        onehot, pof, (((1,), (0,)), ((), ())),
        preferred_element_type=jnp.float32)
    rank_sel = jnp.sum(onehot * rank, axis=1, keepdims=True)
    pos_ref[...] = (pos_sel + rank_sel).astype(jnp.int32)
    po_ref[...] = pof.astype(jnp.int32)
    nt_ref[...] = ntiles


def _router(xf, rw, rb, interpret=False):
    return pl.pallas_call(
        _router_body,
        out_shape=(jax.ShapeDtypeStruct((T, 1), jnp.int32),
                   jax.ShapeDtypeStruct((E, 1), jnp.int32),
                   jax.ShapeDtypeStruct((E, 1), jnp.int32)),
        interpret=interpret,
    )(xf, rw, rb.reshape(1, E))


def _copy_body(pos_ref, src_ref, dst_ref):
    dst_ref[...] = src_ref[...]


_NC = 2
_NS = 16
_NW = _NC * _NS
_BPW = T // _NW


@functools.lru_cache(maxsize=None)
def _sc_dispatch_kernels():
    mesh = plsc.VectorSubcoreMesh(core_axis_name="c", subcore_axis_name="s")
    scratch = [
        pltpu.VMEM((_BPW,), jnp.int32),
        pltpu.VMEM((_BPW, H), jnp.float32),
        pltpu.SemaphoreType.DMA,
    ]

    @functools.partial(
        pl.kernel, mesh=mesh,
        out_type=jax.ShapeDtypeStruct((TPAD, H), jnp.float32),
        scratch_types=scratch)
    def sc_scatter(pos_hbm, x_hbm, out_hbm, idx_v, rows_v, sem):
        wid = lax.axis_index("s") * _NC + lax.axis_index("c")
        base = wid * _BPW
        pltpu.sync_copy(pos_hbm.at[pl.ds(base, _BPW)], idx_v)
        pltpu.sync_copy(x_hbm.at[pl.ds(base, _BPW)], rows_v)
        pltpu.async_copy(rows_v, out_hbm.at[idx_v], sem).wait()

    @functools.partial(
        pl.kernel, mesh=mesh,
        out_type=jax.ShapeDtypeStruct((T, H // 2), jnp.int32),
        scratch_types=[
            pltpu.VMEM((_BPW,), jnp.int32),
            pltpu.VMEM((_BPW, H // 2), jnp.int32),
            pltpu.SemaphoreType.DMA,
        ])
    def sc_gather(pos_hbm, ffn_hbm, out_hbm, idx_v, rows_v, sem):
        wid = lax.axis_index("s") * _NC + lax.axis_index("c")
        base = wid * _BPW
        pltpu.sync_copy(pos_hbm.at[pl.ds(base, _BPW)], idx_v)
        pltpu.async_copy(ffn_hbm.at[idx_v], rows_v, sem).wait()
        pltpu.sync_copy(rows_v, out_hbm.at[pl.ds(base, _BPW)])

    return sc_scatter, sc_gather


def _sc_scatter(pos, xf):
    return _sc_dispatch_kernels()[0](pos, xf)


def _sc_gather(pos, ffn):
    ffn_i32 = jax.lax.bitcast_convert_type(
        ffn.reshape(TPAD, H // 2, 2), jnp.int32)
    out_i32 = _sc_dispatch_kernels()[1](pos, ffn_i32)
    return jax.lax.bitcast_convert_type(
        out_i32.reshape(T, H // 2, 1), jnp.bfloat16).reshape(T, H)


def _scatter_rows(pos, xf, interpret=False):
    grid_spec = pltpu.PrefetchScalarGridSpec(
        num_scalar_prefetch=1,
        grid=(T,),
        in_specs=[pl.BlockSpec((1, 1, H), lambda t, pos: (t, 0, 0))],
        out_specs=pl.BlockSpec((1, 1, H), lambda t, pos: (pos[t], 0, 0)),
    )
    out = pl.pallas_call(
        _copy_body,
        grid_spec=grid_spec,
        out_shape=jax.ShapeDtypeStruct((TPAD, 1, H), jnp.float32),
        interpret=interpret,
    )(pos, xf.reshape(T, 1, H))
    return out.reshape(TPAD, H)


def _gather_rows(pos, ffn, interpret=False):
    grid_spec = pltpu.PrefetchScalarGridSpec(
        num_scalar_prefetch=1,
        grid=(T,),
        in_specs=[pl.BlockSpec((1, 1, H), lambda t, pos: (pos[t], 0, 0))],
        out_specs=pl.BlockSpec((1, 1, H), lambda t, pos: (t, 0, 0)),
    )
    out = pl.pallas_call(
        _copy_body,
        grid_spec=grid_spec,
        out_shape=jax.ShapeDtypeStruct((T, 1, H), ffn.dtype),
        interpret=interpret,
    )(pos, ffn.reshape(TPAD, 1, H))
    return out.reshape(T, H)


FH = F // 2


def _fused_body(x_ref, rw_ref, rb_ref, w1_ref, b1_ref,
                w2_ref, b2_ref, o_ref, pos_out_ref,
                xs_ref, metav_ref):
    e = pl.program_id(0)

    @pl.when(e == 0)
    def _():
        x = x_ref[...]
        logits = jax.lax.dot_general(
            x, rw_ref[...], (((1,), (1,)), ((), ())),
            preferred_element_type=jnp.float32) + rb_ref[...]
        e_iota = jax.lax.broadcasted_iota(jnp.int32, (T, E), 1)
        m = jnp.max(logits, axis=1, keepdims=True)
        eid = jnp.min(jnp.where(logits == m, e_iota, E), axis=1,
                      keepdims=True)
        onehot = (e_iota == eid).astype(jnp.float32)
        r_i = jax.lax.broadcasted_iota(jnp.int32, (T, T), 0)
        c_i = jax.lax.broadcasted_iota(jnp.int32, (T, T), 1)
        lt = (c_i < r_i).astype(jnp.float32)
        rank = jax.lax.dot_general(
            lt, onehot, (((1,), (0,)), ((), ())),
            preferred_element_type=jnp.float32)
        ones_col = jnp.full((T, 1), 1.0, dtype=jnp.float32)
        cntf = jax.lax.dot_general(
            onehot, ones_col, (((0,), (0,)), ((), ())),
            preferred_element_type=jnp.float32)
        ntiles = (cntf.astype(jnp.int32) + (TM - 1)) // TM
        pcf = (ntiles * TM).astype(jnp.float32)
        ei = jax.lax.broadcasted_iota(jnp.int32, (E, E), 0)
        ej = jax.lax.broadcasted_iota(jnp.int32, (E, E), 1)
        ltE = (ej < ei).astype(jnp.float32)
        pof = jax.lax.dot_general(
            ltE, pcf, (((1,), (0,)), ((), ())),
            preferred_element_type=jnp.float32)
        pos_sel = jax.lax.dot_general(
            onehot, pof, (((1,), (0,)), ((), ())),
            preferred_element_type=jnp.float32)
        rank_sel = jnp.sum(onehot * rank, axis=1, keepdims=True)
        posi = (pos_sel + rank_sel).astype(jnp.int32)
        pos_out_ref[...] = posi
        metav_ref[...] = jnp.concatenate(
            [pof.astype(jnp.int32), ntiles], axis=1)

        def scat(t, carry):
            dst = pos_out_ref[t, 0]
            xs_ref[pl.ds(dst, 1), :] = x_ref[pl.ds(t, 1), :]
            return carry

        jax.lax.fori_loop(0, T, scat, 0)

    base = metav_ref[e, 0]
    ntil = metav_ref[e, 1]

    def tile(mi, carry):
        r0 = pl.multiple_of(base + mi * TM, TM)
        xt = xs_ref[pl.ds(r0, TM), :]
        h = jax.lax.dot_general(
            xt, w1_ref[0], (((1,), (1,)), ((), ())),
            preferred_element_type=jnp.float32)
        h = jnp.maximum(h + b1_ref[0], 0.0)
        o = jax.lax.dot_general(
            h, w2_ref[0], (((1,), (1,)), ((), ())),
            preferred_element_type=jnp.float32)
        o_ref[pl.ds(r0, TM), :] = (o + b2_ref[0]).astype(jnp.bfloat16)
        return carry

    jax.lax.fori_loop(0, ntil, tile, 0)


def _fused_ffn(xf, rw, rb, w1, b1, w2, b2, interpret=False):
    out, pos2d = pl.pallas_call(
        _fused_body,
        grid=(E,),
        in_specs=[
            pl.BlockSpec((T, H), lambda e: (0, 0)),
            pl.BlockSpec((E, H), lambda e: (0, 0)),
            pl.BlockSpec((1, E), lambda e: (0, 0)),
            pl.BlockSpec((1, F, H), lambda e: (e, 0, 0)),
            pl.BlockSpec((1, 1, F), lambda e: (e, 0, 0)),
            pl.BlockSpec((1, H, F), lambda e: (e, 0, 0)),
            pl.BlockSpec((1, 1, H), lambda e: (e, 0, 0)),
        ],
        out_specs=(pl.BlockSpec((TPAD, H), lambda e: (0, 0)),
                   pl.BlockSpec((T, 1), lambda e: (0, 0))),
        out_shape=(jax.ShapeDtypeStruct((TPAD, H), jnp.bfloat16),
                   jax.ShapeDtypeStruct((T, 1), jnp.int32)),
        scratch_shapes=[
            pltpu.VMEM((TPAD, H), jnp.float32),
            pltpu.VMEM((E, 2), jnp.int32),
        ],
        compiler_params=pltpu.CompilerParams(
            vmem_limit_bytes=63 * 1024 * 1024),
        interpret=interpret,
    )(xf, rw, rb.reshape(1, E), w1, b1.reshape(E, 1, F),
      w2, b2.reshape(E, 1, H))
    return out, pos2d


def _ffn_body(po_ref, nt_ref, x_ref, w1a_ref, w1b_ref, b1_ref,
              w2a_ref, w2b_ref, b2_ref, o_ref):
    e = pl.program_id(0)
    base = po_ref[e]
    ntiles = nt_ref[e]

    def tile(mi, carry):
        r0 = pl.multiple_of(base + mi * TM, TM)
        xt = x_ref[pl.ds(r0, TM), :]
        ha = jax.lax.dot_general(
            xt, w1a_ref[0], (((1,), (1,)), ((), ())),
            preferred_element_type=jnp.float32)
        hb = jax.lax.dot_general(
            xt, w1b_ref[0], (((1,), (1,)), ((), ())),
            preferred_element_type=jnp.float32)
        b1 = b1_ref[0]
        ha = jnp.maximum(ha + b1[:, :FH], 0.0)
        hb = jnp.maximum(hb + b1[:, FH:], 0.0)
        o = jax.lax.dot_general(
            ha, w2a_ref[0], (((1,), (1,)), ((), ())),
            preferred_element_type=jnp.float32)
        o = o + jax.lax.dot_general(
            hb, w2b_ref[0], (((1,), (1,)), ((), ())),
            preferred_element_type=jnp.float32)
        o_ref[pl.ds(r0, TM), :] = o + b2_ref[0]
        return carry

    jax.lax.fori_loop(0, ntiles, tile, 0)


def _ffn(po, nt, x_sorted, w1, b1, w2, b2, interpret=False):
    grid_spec = pltpu.PrefetchScalarGridSpec(
        num_scalar_prefetch=2,
        grid=(E,),
        in_specs=[
            pl.BlockSpec((TPAD, H), lambda e, po, nt: (0, 0)),
            pl.BlockSpec((1, FH, H), lambda e, po, nt: (e, 0, 0)),
            pl.BlockSpec((1, FH, H), lambda e, po, nt: (e, 1, 0)),
            pl.BlockSpec((1, 1, F), lambda e, po, nt: (e, 0, 0)),
            pl.BlockSpec((1, H, FH), lambda e, po, nt: (e, 0, 0)),
            pl.BlockSpec((1, H, FH), lambda e, po, nt: (e, 0, 1)),
            pl.BlockSpec((1, 1, H), lambda e, po, nt: (e, 0, 0)),
        ],
        out_specs=pl.BlockSpec((TPAD, H), lambda e, po, nt: (0, 0)),
    )
    return pl.pallas_call(
        _ffn_body,
        grid_spec=grid_spec,
        out_shape=jax.ShapeDtypeStruct((TPAD, H), jnp.float32),
        compiler_params=pltpu.CompilerParams(
            vmem_limit_bytes=100 * 1024 * 1024),
        interpret=interpret,
    )(po, nt, x_sorted, w1, w1, b1.reshape(E, 1, F),
      w2, w2, b2.reshape(E, 1, H))


def _moe(x, router_w, router_b, w1, b1, w2, b2, interpret=False):
    B, S, HH = x.shape
    xf = x.reshape(S, HH)
    ffn, pos2d = _fused_ffn(xf, router_w, router_b, w1, b1, w2, b2,
                            interpret=interpret)
    pos = pos2d.reshape(T)
    if interpret:
        out = _gather_rows(pos, ffn, interpret=True)
    else:
        out = _sc_gather(pos, ffn)
    return out.astype(jnp.float32).reshape(B, S, HH)


def kernel(x, router_w, router_b, w1, b1, w2, b2):
    return _moe(x, router_w, router_b, w1, b1, w2, b2)


# speedup vs baseline: 1.2893x; 1.2893x over previous
import functools

import jax
import jax.numpy as jnp
from jax import lax
from jax.experimental import pallas as pl
from jax.experimental.pallas import tpu as pltpu
from jax.experimental.pallas import tpu_sc as plsc

H = 768
F = 4 * H
E = 8
TM = 128
T = 2048
NT = T // TM + E
TPAD = NT * TM
NTE = 32


def _router_body(x_ref, rw_ref, rb_ref, pos_ref, po_ref, nt_ref):
    x = x_ref[...]
    rw = rw_ref[...]
    rb = rb_ref[...]
    logits = jax.lax.dot_general(
        x, rw, (((1,), (1,)), ((), ())),
        preferred_element_type=jnp.float32) + rb
    e_iota = jax.lax.broadcasted_iota(jnp.int32, (T, E), 1)
    m = jnp.max(logits, axis=1, keepdims=True)
    eid = jnp.min(jnp.where(logits == m, e_iota, E), axis=1, keepdims=True)
    onehot = (e_iota == eid).astype(jnp.float32)
    r_i = jax.lax.broadcasted_iota(jnp.int32, (T, T), 0)
    c_i = jax.lax.broadcasted_iota(jnp.int32, (T, T), 1)
    lt = (c_i < r_i).astype(jnp.float32)
    rank = jax.lax.dot_general(
        lt, onehot, (((1,), (0,)), ((), ())),
        preferred_element_type=jnp.float32)
    ones_col = jnp.full((T, 1), 1.0, dtype=jnp.float32)
    cntf = jax.lax.dot_general(
        onehot, ones_col, (((0,), (0,)), ((), ())),
        preferred_element_type=jnp.float32)
    ntiles = (cntf.astype(jnp.int32) + (TM - 1)) // TM
    pcf = (ntiles * TM).astype(jnp.float32)
    ei = jax.lax.broadcasted_iota(jnp.int32, (E, E), 0)
    ej = jax.lax.broadcasted_iota(jnp.int32, (E, E), 1)
    ltE = (ej < ei).astype(jnp.float32)
    pof = jax.lax.dot_general(
        ltE, pcf, (((1,), (0,)), ((), ())),
        preferred_element_type=jnp.float32)
    pos_sel = jax.lax.dot_general(
        onehot, pof, (((1,), (0,)), ((), ())),
        preferred_element_type=jnp.float32)
    rank_sel = jnp.sum(onehot * rank, axis=1, keepdims=True)
    pos_ref[...] = (pos_sel + rank_sel).astype(jnp.int32)
    po_ref[...] = pof.astype(jnp.int32)
    nt_ref[...] = ntiles


def _router(xf, rw, rb, interpret=False):
    return pl.pallas_call(
        _router_body,
        out_shape=(jax.ShapeDtypeStruct((T, 1), jnp.int32),
                   jax.ShapeDtypeStruct((E, 1), jnp.int32),
                   jax.ShapeDtypeStruct((E, 1), jnp.int32)),
        interpret=interpret,
    )(xf, rw, rb.reshape(1, E))


def _copy_body(pos_ref, src_ref, dst_ref):
    dst_ref[...] = src_ref[...]


_NC = 2
_NS = 16
_NW = _NC * _NS
_BPW = T // _NW


@functools.lru_cache(maxsize=None)
def _sc_dispatch_kernels():
    mesh = plsc.VectorSubcoreMesh(core_axis_name="c", subcore_axis_name="s")
    scratch = [
        pltpu.VMEM((_BPW,), jnp.int32),
        pltpu.VMEM((_BPW, H), jnp.float32),
        pltpu.SemaphoreType.DMA,
    ]

    @functools.partial(
        pl.kernel, mesh=mesh,
        out_type=jax.ShapeDtypeStruct((TPAD, H), jnp.float32),
        scratch_types=scratch)
    def sc_scatter(pos_hbm, x_hbm, out_hbm, idx_v, rows_v, sem):
        wid = lax.axis_index("s") * _NC + lax.axis_index("c")
        base = wid * _BPW
        pltpu.sync_copy(pos_hbm.at[pl.ds(base, _BPW)], idx_v)
        pltpu.sync_copy(x_hbm.at[pl.ds(base, _BPW)], rows_v)
        pltpu.async_copy(rows_v, out_hbm.at[idx_v], sem).wait()

    @functools.partial(
        pl.kernel, mesh=mesh,
        out_type=jax.ShapeDtypeStruct((T, H // 2), jnp.int32),
        scratch_types=[
            pltpu.VMEM((_BPW,), jnp.int32),
            pltpu.VMEM((_BPW, H // 2), jnp.int32),
            pltpu.SemaphoreType.DMA,
        ])
    def sc_gather(pos_hbm, ffn_hbm, out_hbm, idx_v, rows_v, sem):
        wid = lax.axis_index("s") * _NC + lax.axis_index("c")
        base = wid * _BPW
        pltpu.sync_copy(pos_hbm.at[pl.ds(base, _BPW)], idx_v)
        pltpu.async_copy(ffn_hbm.at[idx_v], rows_v, sem).wait()
        pltpu.sync_copy(rows_v, out_hbm.at[pl.ds(base, _BPW)])

    return sc_scatter, sc_gather


def _sc_scatter(pos, xf):
    return _sc_dispatch_kernels()[0](pos, xf)


def _sc_gather(pos, ffn):
    ffn_i32 = jax.lax.bitcast_convert_type(
        ffn.reshape(TPAD, H // 2, 2), jnp.int32)
    out_i32 = _sc_dispatch_kernels()[1](pos, ffn_i32)
    return jax.lax.bitcast_convert_type(
        out_i32.reshape(T, H // 2, 1), jnp.bfloat16).reshape(T, H)


def _scatter_rows(pos, xf, interpret=False):
    grid_spec = pltpu.PrefetchScalarGridSpec(
        num_scalar_prefetch=1,
        grid=(T,),
        in_specs=[pl.BlockSpec((1, 1, H), lambda t, pos: (t, 0, 0))],
        out_specs=pl.BlockSpec((1, 1, H), lambda t, pos: (pos[t], 0, 0)),
    )
    out = pl.pallas_call(
        _copy_body,
        grid_spec=grid_spec,
        out_shape=jax.ShapeDtypeStruct((TPAD, 1, H), jnp.float32),
        interpret=interpret,
    )(pos, xf.reshape(T, 1, H))
    return out.reshape(TPAD, H)


def _gather_rows(pos, ffn, interpret=False):
    grid_spec = pltpu.PrefetchScalarGridSpec(
        num_scalar_prefetch=1,
        grid=(T,),
        in_specs=[pl.BlockSpec((1, 1, H), lambda t, pos: (pos[t], 0, 0))],
        out_specs=pl.BlockSpec((1, 1, H), lambda t, pos: (t, 0, 0)),
    )
    out = pl.pallas_call(
        _copy_body,
        grid_spec=grid_spec,
        out_shape=jax.ShapeDtypeStruct((T, 1, H), ffn.dtype),
        interpret=interpret,
    )(pos, ffn.reshape(TPAD, 1, H))
    return out.reshape(T, H)


FH = F // 2


def _fused_body(pos_ref, po_ref, nt_ref, x_ref, w1_ref, b1_ref,
                w2_ref, b2_ref, o_ref, xs_ref):
    e = pl.program_id(0)

    @pl.when(e == 0)
    def _():
        def scat(t, carry):
            dst = pos_ref[t]
            xs_ref[pl.ds(dst, 1), :] = x_ref[pl.ds(t, 1), :]
            return carry

        jax.lax.fori_loop(0, T, scat, 0, unroll=8)

    base = po_ref[e]
    ntil = nt_ref[e]

    def tile(mi, carry):
        r0 = pl.multiple_of(base + mi * TM, TM)
        xt = xs_ref[pl.ds(r0, TM), :]
        h = jax.lax.dot_general(
            xt, w1_ref[0], (((1,), (1,)), ((), ())),
            preferred_element_type=jnp.float32)
        h = jnp.maximum(h + b1_ref[0], 0.0)
        o = jax.lax.dot_general(
            h, w2_ref[0], (((1,), (1,)), ((), ())),
            preferred_element_type=jnp.float32)
        o_ref[pl.ds(r0, TM), :] = (o + b2_ref[0]).astype(jnp.bfloat16)
        return carry

    jax.lax.fori_loop(0, ntil, tile, 0)


def _fused_ffn(pos, po, nt, xf, w1, b1, w2, b2, interpret=False):
    grid_spec = pltpu.PrefetchScalarGridSpec(
        num_scalar_prefetch=3,
        grid=(E,),
        in_specs=[
            pl.BlockSpec((T, H), lambda e, pos, po, nt: (0, 0)),
            pl.BlockSpec((1, F, H), lambda e, pos, po, nt: (e, 0, 0)),
            pl.BlockSpec((1, 1, F), lambda e, pos, po, nt: (e, 0, 0)),
            pl.BlockSpec((1, H, F), lambda e, pos, po, nt: (e, 0, 0)),
            pl.BlockSpec((1, 1, H), lambda e, pos, po, nt: (e, 0, 0)),
        ],
        out_specs=pl.BlockSpec((TPAD, H), lambda e, pos, po, nt: (0, 0)),
        scratch_shapes=[
            pltpu.VMEM((TPAD, H), jnp.float32),
        ],
    )
    return pl.pallas_call(
        _fused_body,
        grid_spec=grid_spec,
        out_shape=jax.ShapeDtypeStruct((TPAD, H), jnp.bfloat16),
        compiler_params=pltpu.CompilerParams(
            vmem_limit_bytes=63 * 1024 * 1024),
        interpret=interpret,
    )(pos, po, nt, xf, w1, b1.reshape(E, 1, F),
      w2, b2.reshape(E, 1, H))


def _ffn_body(po_ref, nt_ref, x_ref, w1a_ref, w1b_ref, b1_ref,
              w2a_ref, w2b_ref, b2_ref, o_ref):
    e = pl.program_id(0)
    base = po_ref[e]
    ntiles = nt_ref[e]

    def tile(mi, carry):
        r0 = pl.multiple_of(base + mi * TM, TM)
        xt = x_ref[pl.ds(r0, TM), :]
        ha = jax.lax.dot_general(
            xt, w1a_ref[0], (((1,), (1,)), ((), ())),
            preferred_element_type=jnp.float32)
        hb = jax.lax.dot_general(
            xt, w1b_ref[0], (((1,), (1,)), ((), ())),
            preferred_element_type=jnp.float32)
        b1 = b1_ref[0]
        ha = jnp.maximum(ha + b1[:, :FH], 0.0)
        hb = jnp.maximum(hb + b1[:, FH:], 0.0)
        o = jax.lax.dot_general(
            ha, w2a_ref[0], (((1,), (1,)), ((), ())),
            preferred_element_type=jnp.float32)
        o = o + jax.lax.dot_general(
            hb, w2b_ref[0], (((1,), (1,)), ((), ())),
            preferred_element_type=jnp.float32)
        o_ref[pl.ds(r0, TM), :] = o + b2_ref[0]
        return carry

    jax.lax.fori_loop(0, ntiles, tile, 0)


def _ffn(po, nt, x_sorted, w1, b1, w2, b2, interpret=False):
    grid_spec = pltpu.PrefetchScalarGridSpec(
        num_scalar_prefetch=2,
        grid=(E,),
        in_specs=[
            pl.BlockSpec((TPAD, H), lambda e, po, nt: (0, 0)),
            pl.BlockSpec((1, FH, H), lambda e, po, nt: (e, 0, 0)),
            pl.BlockSpec((1, FH, H), lambda e, po, nt: (e, 1, 0)),
            pl.BlockSpec((1, 1, F), lambda e, po, nt: (e, 0, 0)),
            pl.BlockSpec((1, H, FH), lambda e, po, nt: (e, 0, 0)),
            pl.BlockSpec((1, H, FH), lambda e, po, nt: (e, 0, 1)),
            pl.BlockSpec((1, 1, H), lambda e, po, nt: (e, 0, 0)),
        ],
        out_specs=pl.BlockSpec((TPAD, H), lambda e, po, nt: (0, 0)),
    )
    return pl.pallas_call(
        _ffn_body,
        grid_spec=grid_spec,
        out_shape=jax.ShapeDtypeStruct((TPAD, H), jnp.float32),
        compiler_params=pltpu.CompilerParams(
            vmem_limit_bytes=100 * 1024 * 1024),
        interpret=interpret,
    )(po, nt, x_sorted, w1, w1, b1.reshape(E, 1, F),
      w2, w2, b2.reshape(E, 1, H))


def _moe(x, router_w, router_b, w1, b1, w2, b2, interpret=False):
    B, S, HH = x.shape
    xf = x.reshape(S, HH)
    pos2d, po2d, nt2d = _router(xf, router_w, router_b, interpret=interpret)
    pos = pos2d.reshape(T)
    po = po2d.reshape(E)
    nt = nt2d.reshape(E)
    ffn = _fused_ffn(pos, po, nt, xf, w1, b1, w2, b2, interpret=interpret)
    if interpret:
        out = _gather_rows(pos, ffn, interpret=True)
    else:
        out = _sc_gather(pos, ffn)
    return out.astype(jnp.float32).reshape(B, S, HH)


def kernel(x, router_w, router_b, w1, b1, w2, b2):
    return _moe(x, router_w, router_b, w1, b1, w2, b2)


# speedup vs baseline: 2.2129x; 1.7163x over previous
import functools

import jax
import jax.numpy as jnp
from jax import lax
from jax.experimental import pallas as pl
from jax.experimental.pallas import tpu as pltpu
from jax.experimental.pallas import tpu_sc as plsc

H = 768
F = 4 * H
E = 8
TM = 128
T = 2048
NT = T // TM + E
TPAD = NT * TM
NTE = 32


def _router_body(x_ref, rw_ref, rb_ref, pos_ref, po_ref, nt_ref):
    x = x_ref[...]
    rw = rw_ref[...]
    rb = rb_ref[...]
    logits = jax.lax.dot_general(
        x, rw, (((1,), (1,)), ((), ())),
        preferred_element_type=jnp.float32) + rb
    e_iota = jax.lax.broadcasted_iota(jnp.int32, (T, E), 1)
    m = jnp.max(logits, axis=1, keepdims=True)
    eid = jnp.min(jnp.where(logits == m, e_iota, E), axis=1, keepdims=True)
    onehot = (e_iota == eid).astype(jnp.float32)
    r_i = jax.lax.broadcasted_iota(jnp.int32, (T, T), 0)
    c_i = jax.lax.broadcasted_iota(jnp.int32, (T, T), 1)
    lt = (c_i < r_i).astype(jnp.float32)
    rank = jax.lax.dot_general(
        lt, onehot, (((1,), (0,)), ((), ())),
        preferred_element_type=jnp.float32)
    ones_col = jnp.full((T, 1), 1.0, dtype=jnp.float32)
    cntf = jax.lax.dot_general(
        onehot, ones_col, (((0,), (0,)), ((), ())),
        preferred_element_type=jnp.float32)
    ntiles = (cntf.astype(jnp.int32) + (TM - 1)) // TM
    pcf = (ntiles * TM).astype(jnp.float32)
    ei = jax.lax.broadcasted_iota(jnp.int32, (E, E), 0)
    ej = jax.lax.broadcasted_iota(jnp.int32, (E, E), 1)
    ltE = (ej < ei).astype(jnp.float32)
    pof = jax.lax.dot_general(
        ltE, pcf, (((1,), (0,)), ((), ())),
        preferred_element_type=jnp.float32)
    pos_sel = jax.lax.dot_general(
        onehot, pof, (((1,), (0,)), ((), ())),
        preferred_element_type=jnp.float32)
    rank_sel = jnp.sum(onehot * rank, axis=1, keepdims=True)
    pos_ref[...] = (pos_sel + rank_sel).astype(jnp.int32)
    po_ref[...] = pof.astype(jnp.int32)
    nt_ref[...] = ntiles


def _router(xf, rw, rb, interpret=False):
    return pl.pallas_call(
        _router_body,
        out_shape=(jax.ShapeDtypeStruct((T, 1), jnp.int32),
                   jax.ShapeDtypeStruct((E, 1), jnp.int32),
                   jax.ShapeDtypeStruct((E, 1), jnp.int32)),
        interpret=interpret,
    )(xf, rw, rb.reshape(1, E))


def _copy_body(pos_ref, src_ref, dst_ref):
    dst_ref[...] = src_ref[...]


_NC = 2
_NS = 16
_NW = _NC * _NS
_BPW = T // _NW


@functools.lru_cache(maxsize=None)
def _sc_dispatch_kernels():
    @functools.partial(
        pl.kernel,
        mesh=plsc.VectorSubcoreMesh(core_axis_name="c", subcore_axis_name="s"),
        out_type=jax.ShapeDtypeStruct((TPAD, H), jnp.float32),
        scratch_types=[
            pltpu.VMEM((_BPW,), jnp.int32),
            pltpu.VMEM((_BPW, H), jnp.float32),
            pltpu.SemaphoreType.DMA,
        ])
    def sc_scatter(pos_hbm, x_hbm, out_hbm, idx_v, rows_v, sem):
        wid = lax.axis_index("s") * _NC + lax.axis_index("c")
        base = wid * _BPW
        pltpu.sync_copy(pos_hbm.at[pl.ds(base, _BPW)], idx_v)
        pltpu.sync_copy(x_hbm.at[pl.ds(base, _BPW)], rows_v)
        pltpu.async_copy(rows_v, out_hbm.at[idx_v], sem).wait()

    @functools.partial(
        pl.kernel,
        mesh=plsc.VectorSubcoreMesh(core_axis_name="c", subcore_axis_name="s"),
        out_type=jax.ShapeDtypeStruct((T, H), jnp.float32),
        scratch_types=[
            pltpu.VMEM((_BPW,), jnp.int32),
            pltpu.VMEM((_BPW, H), jnp.float32),
            pltpu.SemaphoreType.DMA,
        ])
    def sc_gather(pos_hbm, ffn_hbm, out_hbm, idx_v, rows_v, sem):
        wid = lax.axis_index("s") * _NC + lax.axis_index("c")
        base = wid * _BPW
        pltpu.sync_copy(pos_hbm.at[pl.ds(base, _BPW)], idx_v)
        pltpu.async_copy(ffn_hbm.at[idx_v], rows_v, sem).wait()
        pltpu.sync_copy(rows_v, out_hbm.at[pl.ds(base, _BPW)])

    return sc_scatter, sc_gather


def _sc_scatter(pos, xf):
    return _sc_dispatch_kernels()[0](pos, xf)


def _sc_gather(pos, ffn):
    return _sc_dispatch_kernels()[1](pos, ffn)


def _scatter_rows(pos, xf, interpret=False):
    grid_spec = pltpu.PrefetchScalarGridSpec(
        num_scalar_prefetch=1,
        grid=(T,),
        in_specs=[pl.BlockSpec((1, 1, H), lambda t, pos: (t, 0, 0))],
        out_specs=pl.BlockSpec((1, 1, H), lambda t, pos: (pos[t], 0, 0)),
    )
    out = pl.pallas_call(
        _copy_body,
        grid_spec=grid_spec,
        out_shape=jax.ShapeDtypeStruct((TPAD, 1, H), jnp.float32),
        interpret=interpret,
    )(pos, xf.reshape(T, 1, H))
    return out.reshape(TPAD, H)


def _gather_rows(pos, ffn, interpret=False):
    grid_spec = pltpu.PrefetchScalarGridSpec(
        num_scalar_prefetch=1,
        grid=(T,),
        in_specs=[pl.BlockSpec((1, 1, H), lambda t, pos: (pos[t], 0, 0))],
        out_specs=pl.BlockSpec((1, 1, H), lambda t, pos: (t, 0, 0)),
    )
    out = pl.pallas_call(
        _copy_body,
        grid_spec=grid_spec,
        out_shape=jax.ShapeDtypeStruct((T, 1, H), ffn.dtype),
        interpret=interpret,
    )(pos, ffn.reshape(TPAD, 1, H))
    return out.reshape(T, H)


FH = F // 2


def _ffn_body(po_ref, nt_ref, x_ref, w1_ref, b1_ref,
              w2_ref, b2_ref, o_ref):
    e = pl.program_id(0)
    base = po_ref[e]
    ntil = nt_ref[e]

    def tile(mi, carry):
        r0 = pl.multiple_of(base + mi * TM, TM)
        xt = x_ref[pl.ds(r0, TM), :]
        h = jax.lax.dot_general(
            xt, w1_ref[0], (((1,), (1,)), ((), ())),
            preferred_element_type=jnp.float32)
        h = jnp.maximum(h + b1_ref[0], 0.0)
        o = jax.lax.dot_general(
            h, w2_ref[0], (((1,), (1,)), ((), ())),
            preferred_element_type=jnp.float32)
        o_ref[pl.ds(r0, TM), :] = o + b2_ref[0]
        return carry

    jax.lax.fori_loop(0, ntil, tile, 0)


def _ffn(po, nt, x_sorted, w1, b1, w2, b2, interpret=False):
    grid_spec = pltpu.PrefetchScalarGridSpec(
        num_scalar_prefetch=2,
        grid=(E,),
        in_specs=[
            pl.BlockSpec((TPAD, H), lambda e, po, nt: (0, 0)),
            pl.BlockSpec((1, F, H), lambda e, po, nt: (e, 0, 0)),
            pl.BlockSpec((1, 1, F), lambda e, po, nt: (e, 0, 0)),
            pl.BlockSpec((1, H, F), lambda e, po, nt: (e, 0, 0)),
            pl.BlockSpec((1, 1, H), lambda e, po, nt: (e, 0, 0)),
        ],
        out_specs=pl.BlockSpec((TPAD, H), lambda e, po, nt: (0, 0)),
    )
    return pl.pallas_call(
        _ffn_body,
        grid_spec=grid_spec,
        out_shape=jax.ShapeDtypeStruct((TPAD, H), jnp.float32),
        compiler_params=pltpu.CompilerParams(
            vmem_limit_bytes=63 * 1024 * 1024),
        interpret=interpret,
    )(po, nt, x_sorted, w1, b1.reshape(E, 1, F),
      w2, b2.reshape(E, 1, H))


def _ffn_body(po_ref, nt_ref, x_ref, w1a_ref, w1b_ref, b1_ref,
              w2a_ref, w2b_ref, b2_ref, o_ref):
    e = pl.program_id(0)
    base = po_ref[e]
    ntiles = nt_ref[e]

    def tile(mi, carry):
        r0 = pl.multiple_of(base + mi * TM, TM)
        xt = x_ref[pl.ds(r0, TM), :]
        ha = jax.lax.dot_general(
            xt, w1a_ref[0], (((1,), (1,)), ((), ())),
            preferred_element_type=jnp.float32)
        hb = jax.lax.dot_general(
            xt, w1b_ref[0], (((1,), (1,)), ((), ())),
            preferred_element_type=jnp.float32)
        b1 = b1_ref[0]
        ha = jnp.maximum(ha + b1[:, :FH], 0.0)
        hb = jnp.maximum(hb + b1[:, FH:], 0.0)
        o = jax.lax.dot_general(
            ha, w2a_ref[0], (((1,), (1,)), ((), ())),
            preferred_element_type=jnp.float32)
        o = o + jax.lax.dot_general(
            hb, w2b_ref[0], (((1,), (1,)), ((), ())),
            preferred_element_type=jnp.float32)
        o_ref[pl.ds(r0, TM), :] = o + b2_ref[0]
        return carry

    jax.lax.fori_loop(0, ntiles, tile, 0)


def _ffn(po, nt, x_sorted, w1, b1, w2, b2, interpret=False):
    grid_spec = pltpu.PrefetchScalarGridSpec(
        num_scalar_prefetch=2,
        grid=(E,),
        in_specs=[
            pl.BlockSpec((TPAD, H), lambda e, po, nt: (0, 0)),
            pl.BlockSpec((1, FH, H), lambda e, po, nt: (e, 0, 0)),
            pl.BlockSpec((1, FH, H), lambda e, po, nt: (e, 1, 0)),
            pl.BlockSpec((1, 1, F), lambda e, po, nt: (e, 0, 0)),
            pl.BlockSpec((1, H, FH), lambda e, po, nt: (e, 0, 0)),
            pl.BlockSpec((1, H, FH), lambda e, po, nt: (e, 0, 1)),
            pl.BlockSpec((1, 1, H), lambda e, po, nt: (e, 0, 0)),
        ],
        out_specs=pl.BlockSpec((TPAD, H), lambda e, po, nt: (0, 0)),
    )
    return pl.pallas_call(
        _ffn_body,
        grid_spec=grid_spec,
        out_shape=jax.ShapeDtypeStruct((TPAD, H), jnp.float32),
        compiler_params=pltpu.CompilerParams(
            vmem_limit_bytes=100 * 1024 * 1024),
        interpret=interpret,
    )(po, nt, x_sorted, w1, w1, b1.reshape(E, 1, F),
      w2, w2, b2.reshape(E, 1, H))


def _moe(x, router_w, router_b, w1, b1, w2, b2, interpret=False):
    B, S, HH = x.shape
    xf = x.reshape(S, HH)
    pos2d, po2d, nt2d = _router(xf, router_w, router_b, interpret=interpret)
    pos = pos2d.reshape(T)
    po = po2d.reshape(E)
    nt = nt2d.reshape(E)
    if interpret:
        x_sorted = _scatter_rows(pos, xf, interpret=True)
    else:
        x_sorted = _sc_scatter(pos, xf)
    ffn = _ffn(po, nt, x_sorted, w1, b1, w2, b2, interpret=interpret)
    if interpret:
        out = _gather_rows(pos, ffn, interpret=True)
    else:
        out = _sc_gather(pos, ffn)
    return out.reshape(B, S, HH)


def kernel(x, router_w, router_b, w1, b1, w2, b2):
    return _moe(x, router_w, router_b, w1, b1, w2, b2)


# speedup vs baseline: 2.2343x; 1.0097x over previous
import functools

import jax
import jax.numpy as jnp
from jax import lax
from jax.experimental import pallas as pl
from jax.experimental.pallas import tpu as pltpu
from jax.experimental.pallas import tpu_sc as plsc

H = 768
F = 4 * H
E = 8
TM = 128
T = 2048
NT = T // TM + E
TPAD = NT * TM
NTE = 32


def _router_body(x_ref, rw_ref, rb_ref, pos_ref, po_ref, nt_ref):
    x = x_ref[...]
    rw = rw_ref[...]
    rb = rb_ref[...]
    logits = jax.lax.dot_general(
        x, rw, (((1,), (1,)), ((), ())),
        preferred_element_type=jnp.float32) + rb
    e_iota = jax.lax.broadcasted_iota(jnp.int32, (T, E), 1)
    m = jnp.max(logits, axis=1, keepdims=True)
    eid = jnp.min(jnp.where(logits == m, e_iota, E), axis=1, keepdims=True)
    onehot = (e_iota == eid).astype(jnp.float32)
    r_i = jax.lax.broadcasted_iota(jnp.int32, (T, T), 0)
    c_i = jax.lax.broadcasted_iota(jnp.int32, (T, T), 1)
    lt = (c_i < r_i).astype(jnp.float32)
    rank = jax.lax.dot_general(
        lt, onehot, (((1,), (0,)), ((), ())),
        preferred_element_type=jnp.float32)
    ones_col = jnp.full((T, 1), 1.0, dtype=jnp.float32)
    cntf = jax.lax.dot_general(
        onehot, ones_col, (((0,), (0,)), ((), ())),
        preferred_element_type=jnp.float32)
    ntiles = (cntf.astype(jnp.int32) + (TM - 1)) // TM
    pcf = (ntiles * TM).astype(jnp.float32)
    ei = jax.lax.broadcasted_iota(jnp.int32, (E, E), 0)
    ej = jax.lax.broadcasted_iota(jnp.int32, (E, E), 1)
    ltE = (ej < ei).astype(jnp.float32)
    pof = jax.lax.dot_general(
        ltE, pcf, (((1,), (0,)), ((), ())),
        preferred_element_type=jnp.float32)
    pos_sel = jax.lax.dot_general(
        onehot, pof, (((1,), (0,)), ((), ())),
        preferred_element_type=jnp.float32)
    rank_sel = jnp.sum(onehot * rank, axis=1, keepdims=True)
    pos_ref[...] = (pos_sel + rank_sel).astype(jnp.int32)
    po_ref[...] = pof.astype(jnp.int32)
    nt_ref[...] = ntiles


def _router(xf, rw, rb, interpret=False):
    return pl.pallas_call(
        _router_body,
        out_shape=(jax.ShapeDtypeStruct((T, 1), jnp.int32),
                   jax.ShapeDtypeStruct((E, 1), jnp.int32),
                   jax.ShapeDtypeStruct((E, 1), jnp.int32)),
        interpret=interpret,
    )(xf, rw, rb.reshape(1, E))


def _copy_body(pos_ref, src_ref, dst_ref):
    dst_ref[...] = src_ref[...]


_NC = 2
_NS = 16
_NW = _NC * _NS
_BPW = T // _NW


@functools.lru_cache(maxsize=None)
def _sc_dispatch_kernels():
    @functools.partial(
        pl.kernel,
        mesh=plsc.VectorSubcoreMesh(core_axis_name="c", subcore_axis_name="s"),
        out_type=jax.ShapeDtypeStruct((TPAD, H), jnp.float32),
        scratch_types=[
            pltpu.VMEM((_BPW,), jnp.int32),
            pltpu.VMEM((_BPW, H), jnp.float32),
            pltpu.SemaphoreType.DMA,
        ])
    def sc_scatter(pos_hbm, x_hbm, out_hbm, idx_v, rows_v, sem):
        wid = lax.axis_index("s") * _NC + lax.axis_index("c")
        base = wid * _BPW
        pltpu.sync_copy(pos_hbm.at[pl.ds(base, _BPW)], idx_v)
        pltpu.sync_copy(x_hbm.at[pl.ds(base, _BPW)], rows_v)
        pltpu.async_copy(rows_v, out_hbm.at[idx_v], sem).wait()

    @functools.partial(
        pl.kernel,
        mesh=plsc.VectorSubcoreMesh(core_axis_name="c", subcore_axis_name="s"),
        out_type=jax.ShapeDtypeStruct((T, H), jnp.float32),
        scratch_types=[
            pltpu.VMEM((_BPW,), jnp.int32),
            pltpu.VMEM((_BPW, H), jnp.float32),
            pltpu.SemaphoreType.DMA,
        ])
    def sc_gather(pos_hbm, ffn_hbm, out_hbm, idx_v, rows_v, sem):
        wid = lax.axis_index("s") * _NC + lax.axis_index("c")
        base = wid * _BPW
        pltpu.sync_copy(pos_hbm.at[pl.ds(base, _BPW)], idx_v)
        pltpu.async_copy(ffn_hbm.at[idx_v], rows_v, sem).wait()
        pltpu.sync_copy(rows_v, out_hbm.at[pl.ds(base, _BPW)])

    return sc_scatter, sc_gather


def _sc_scatter(pos, xf):
    return _sc_dispatch_kernels()[0](pos, xf)


def _sc_gather(pos, ffn):
    return _sc_dispatch_kernels()[1](pos, ffn)


def _scatter_rows(pos, xf, interpret=False):
    grid_spec = pltpu.PrefetchScalarGridSpec(
        num_scalar_prefetch=1,
        grid=(T,),
        in_specs=[pl.BlockSpec((1, 1, H), lambda t, pos: (t, 0, 0))],
        out_specs=pl.BlockSpec((1, 1, H), lambda t, pos: (pos[t], 0, 0)),
    )
    out = pl.pallas_call(
        _copy_body,
        grid_spec=grid_spec,
        out_shape=jax.ShapeDtypeStruct((TPAD, 1, H), jnp.float32),
        interpret=interpret,
    )(pos, xf.reshape(T, 1, H))
    return out.reshape(TPAD, H)


def _gather_rows(pos, ffn, interpret=False):
    grid_spec = pltpu.PrefetchScalarGridSpec(
        num_scalar_prefetch=1,
        grid=(T,),
        in_specs=[pl.BlockSpec((1, 1, H), lambda t, pos: (pos[t], 0, 0))],
        out_specs=pl.BlockSpec((1, 1, H), lambda t, pos: (t, 0, 0)),
    )
    out = pl.pallas_call(
        _copy_body,
        grid_spec=grid_spec,
        out_shape=jax.ShapeDtypeStruct((T, 1, H), ffn.dtype),
        interpret=interpret,
    )(pos, ffn.reshape(TPAD, 1, H))
    return out.reshape(T, H)


FH = F // 2


def _ffn_body(po_ref, nt_ref, x_ref, w1_ref, b1_ref,
              w2_ref, b2_ref, o_ref):
    e = pl.program_id(0)
    base = po_ref[e]
    ntil = nt_ref[e]

    def tile(mi, carry):
        r0 = pl.multiple_of(base + mi * TM, TM)
        xt = x_ref[pl.ds(r0, TM), :]
        h = jax.lax.dot_general(
            xt, w1_ref[0], (((1,), (1,)), ((), ())),
            preferred_element_type=jnp.float32)
        h = jnp.maximum(h + b1_ref[0], 0.0)
        o = jax.lax.dot_general(
            h, w2_ref[0], (((1,), (1,)), ((), ())),
            preferred_element_type=jnp.float32)
        o_ref[pl.ds(r0, TM), :] = o + b2_ref[0]
        return carry

    jax.lax.fori_loop(0, ntil, tile, 0)


def _ffn(po, nt, x_sorted, w1, b1, w2, b2, interpret=False):
    grid_spec = pltpu.PrefetchScalarGridSpec(
        num_scalar_prefetch=2,
        grid=(E,),
        in_specs=[
            pl.BlockSpec((TPAD, H), lambda e, po, nt: (0, 0)),
            pl.BlockSpec((1, F, H), lambda e, po, nt: (e, 0, 0)),
            pl.BlockSpec((1, 1, F), lambda e, po, nt: (e, 0, 0)),
            pl.BlockSpec((1, H, F), lambda e, po, nt: (e, 0, 0)),
            pl.BlockSpec((1, 1, H), lambda e, po, nt: (e, 0, 0)),
        ],
        out_specs=pl.BlockSpec((TPAD, H), lambda e, po, nt: (0, 0)),
    )
    return pl.pallas_call(
        _ffn_body,
        grid_spec=grid_spec,
        out_shape=jax.ShapeDtypeStruct((TPAD, H), jnp.float32),
        interpret=interpret,
    )(po, nt, x_sorted, w1, b1.reshape(E, 1, F),
      w2, b2.reshape(E, 1, H))


def _ffn_body(po_ref, nt_ref, x_ref, w1a_ref, w1b_ref, b1_ref,
              w2a_ref, w2b_ref, b2_ref, o_ref):
    e = pl.program_id(0)
    base = po_ref[e]
    ntiles = nt_ref[e]

    def tile(mi, carry):
        r0 = pl.multiple_of(base + mi * TM, TM)
        xt = x_ref[pl.ds(r0, TM), :]
        ha = jax.lax.dot_general(
            xt, w1a_ref[0], (((1,), (1,)), ((), ())),
            preferred_element_type=jnp.float32)
        hb = jax.lax.dot_general(
            xt, w1b_ref[0], (((1,), (1,)), ((), ())),
            preferred_element_type=jnp.float32)
        b1 = b1_ref[0]
        ha = jnp.maximum(ha + b1[:, :FH], 0.0)
        hb = jnp.maximum(hb + b1[:, FH:], 0.0)
        o = jax.lax.dot_general(
            ha, w2a_ref[0], (((1,), (1,)), ((), ())),
            preferred_element_type=jnp.float32)
        o = o + jax.lax.dot_general(
            hb, w2b_ref[0], (((1,), (1,)), ((), ())),
            preferred_element_type=jnp.float32)
        o_ref[pl.ds(r0, TM), :] = o + b2_ref[0]
        return carry

    jax.lax.fori_loop(0, ntiles, tile, 0)


def _ffn(po, nt, x_sorted, w1, b1, w2, b2, interpret=False):
    grid_spec = pltpu.PrefetchScalarGridSpec(
        num_scalar_prefetch=2,
        grid=(E,),
        in_specs=[
            pl.BlockSpec((TPAD, H), lambda e, po, nt: (0, 0)),
            pl.BlockSpec((1, FH, H), lambda e, po, nt: (e, 0, 0)),
            pl.BlockSpec((1, FH, H), lambda e, po, nt: (e, 1, 0)),
            pl.BlockSpec((1, 1, F), lambda e, po, nt: (e, 0, 0)),
            pl.BlockSpec((1, H, FH), lambda e, po, nt: (e, 0, 0)),
            pl.BlockSpec((1, H, FH), lambda e, po, nt: (e, 0, 1)),
            pl.BlockSpec((1, 1, H), lambda e, po, nt: (e, 0, 0)),
        ],
        out_specs=pl.BlockSpec((TPAD, H), lambda e, po, nt: (0, 0)),
    )
    return pl.pallas_call(
        _ffn_body,
        grid_spec=grid_spec,
        out_shape=jax.ShapeDtypeStruct((TPAD, H), jnp.float32),
        compiler_params=pltpu.CompilerParams(
            vmem_limit_bytes=100 * 1024 * 1024),
        interpret=interpret,
    )(po, nt, x_sorted, w1, w1, b1.reshape(E, 1, F),
      w2, w2, b2.reshape(E, 1, H))


def _moe(x, router_w, router_b, w1, b1, w2, b2, interpret=False):
    B, S, HH = x.shape
    xf = x.reshape(S, HH)
    pos2d, po2d, nt2d = _router(xf, router_w, router_b, interpret=interpret)
    pos = pos2d.reshape(T)
    po = po2d.reshape(E)
    nt = nt2d.reshape(E)
    if interpret:
        x_sorted = _scatter_rows(pos, xf, interpret=True)
    else:
        x_sorted = _sc_scatter(pos, xf)
    ffn = _ffn(po, nt, x_sorted, w1, b1, w2, b2, interpret=interpret)
    if interpret:
        out = _gather_rows(pos, ffn, interpret=True)
    else:
        out = _sc_gather(pos, ffn)
    return out.reshape(B, S, HH)


def kernel(x, router_w, router_b, w1, b1, w2, b2):
    return _moe(x, router_w, router_b, w1, b1, w2, b2)


# speedup vs baseline: 2.2707x; 1.0163x over previous
import functools

import jax
import jax.numpy as jnp
from jax import lax
from jax.experimental import pallas as pl
from jax.experimental.pallas import tpu as pltpu
from jax.experimental.pallas import tpu_sc as plsc

H = 768
F = 4 * H
E = 8
TM = 128
T = 2048
NT = T // TM + E
TPAD = NT * TM


def _router_body(x_ref, rw_ref, rb_ref, pos_ref, po_ref, nt_ref):
    x = x_ref[...]
    rw = rw_ref[...]
    rb = rb_ref[...]
    logits = jax.lax.dot_general(
        x, rw, (((1,), (1,)), ((), ())),
        preferred_element_type=jnp.float32) + rb
    e_iota = jax.lax.broadcasted_iota(jnp.int32, (T, E), 1)
    m = jnp.max(logits, axis=1, keepdims=True)
    eid = jnp.min(jnp.where(logits == m, e_iota, E), axis=1, keepdims=True)
    onehot = (e_iota == eid).astype(jnp.float32)
    r_i = jax.lax.broadcasted_iota(jnp.int32, (T, T), 0)
    c_i = jax.lax.broadcasted_iota(jnp.int32, (T, T), 1)
    lt = (c_i < r_i).astype(jnp.float32)
    rank = jax.lax.dot_general(
        lt, onehot, (((1,), (0,)), ((), ())),
        preferred_element_type=jnp.float32)
    ones_col = jnp.full((T, 1), 1.0, dtype=jnp.float32)
    cntf = jax.lax.dot_general(
        onehot, ones_col, (((0,), (0,)), ((), ())),
        preferred_element_type=jnp.float32)
    ntiles = (cntf.astype(jnp.int32) + (TM - 1)) // TM
    pcf = (ntiles * TM).astype(jnp.float32)
    ei = jax.lax.broadcasted_iota(jnp.int32, (E, E), 0)
    ej = jax.lax.broadcasted_iota(jnp.int32, (E, E), 1)
    ltE = (ej < ei).astype(jnp.float32)
    pof = jax.lax.dot_general(
        ltE, pcf, (((1,), (0,)), ((), ())),
        preferred_element_type=jnp.float32)
    pos_sel = jax.lax.dot_general(
        onehot, pof, (((1,), (0,)), ((), ())),
        preferred_element_type=jnp.float32)
    rank_sel = jnp.sum(onehot * rank, axis=1, keepdims=True)
    pos_ref[...] = (pos_sel + rank_sel).astype(jnp.int32)
    po_ref[...] = pof.astype(jnp.int32)
    nt_ref[...] = ntiles


def _router(xf, rw, rb):
    return pl.pallas_call(
        _router_body,
        out_shape=(jax.ShapeDtypeStruct((T, 1), jnp.int32),
                   jax.ShapeDtypeStruct((E, 1), jnp.int32),
                   jax.ShapeDtypeStruct((E, 1), jnp.int32)),
    )(xf, rw, rb.reshape(1, E))


_NC = 2
_NS = 16
_NW = _NC * _NS
_BPW = T // _NW


@functools.lru_cache(maxsize=None)
def _sc_dispatch_kernels():
    @functools.partial(
        pl.kernel,
        mesh=plsc.VectorSubcoreMesh(core_axis_name="c", subcore_axis_name="s"),
        out_type=jax.ShapeDtypeStruct((TPAD, H), jnp.float32),
        scratch_types=[
            pltpu.VMEM((_BPW,), jnp.int32),
            pltpu.VMEM((_BPW, H), jnp.float32),
            pltpu.SemaphoreType.DMA,
        ])
    def sc_scatter(pos_hbm, x_hbm, out_hbm, idx_v, rows_v, sem):
        wid = lax.axis_index("s") * _NC + lax.axis_index("c")
        base = wid * _BPW
        pltpu.sync_copy(pos_hbm.at[pl.ds(base, _BPW)], idx_v)
        pltpu.sync_copy(x_hbm.at[pl.ds(base, _BPW)], rows_v)
        pltpu.async_copy(rows_v, out_hbm.at[idx_v], sem).wait()

    @functools.partial(
        pl.kernel,
        mesh=plsc.VectorSubcoreMesh(core_axis_name="c", subcore_axis_name="s"),
        out_type=jax.ShapeDtypeStruct((T, H), jnp.float32),
        scratch_types=[
            pltpu.VMEM((_BPW,), jnp.int32),
            pltpu.VMEM((_BPW, H), jnp.float32),
            pltpu.SemaphoreType.DMA,
        ])
    def sc_gather(pos_hbm, ffn_hbm, out_hbm, idx_v, rows_v, sem):
        wid = lax.axis_index("s") * _NC + lax.axis_index("c")
        base = wid * _BPW
        pltpu.sync_copy(pos_hbm.at[pl.ds(base, _BPW)], idx_v)
        pltpu.async_copy(ffn_hbm.at[idx_v], rows_v, sem).wait()
        pltpu.sync_copy(rows_v, out_hbm.at[pl.ds(base, _BPW)])

    return sc_scatter, sc_gather


def _ffn_body(po_ref, nt_ref, x_ref, w1_ref, b1_ref,
              w2_ref, b2_ref, o_ref):
    e = pl.program_id(0)
    base = po_ref[e]
    ntil = nt_ref[e]

    def tile(mi, carry):
        r0 = pl.multiple_of(base + mi * TM, TM)
        xt = x_ref[pl.ds(r0, TM), :]
        h = jax.lax.dot_general(
            xt, w1_ref[0], (((1,), (1,)), ((), ())),
            preferred_element_type=jnp.float32)
        h = jnp.maximum(h + b1_ref[0], 0.0)
        o = jax.lax.dot_general(
            h, w2_ref[0], (((1,), (1,)), ((), ())),
            preferred_element_type=jnp.float32)
        o_ref[pl.ds(r0, TM), :] = o + b2_ref[0]
        return carry

    jax.lax.fori_loop(0, ntil, tile, 0)


def _ffn(po, nt, x_sorted, w1, b1, w2, b2):
    grid_spec = pltpu.PrefetchScalarGridSpec(
        num_scalar_prefetch=2,
        grid=(E,),
        in_specs=[
            pl.BlockSpec((TPAD, H), lambda e, po, nt: (0, 0)),
            pl.BlockSpec((1, F, H), lambda e, po, nt: (e, 0, 0)),
            pl.BlockSpec((1, 1, F), lambda e, po, nt: (e, 0, 0)),
            pl.BlockSpec((1, H, F), lambda e, po, nt: (e, 0, 0)),
            pl.BlockSpec((1, 1, H), lambda e, po, nt: (e, 0, 0)),
        ],
        out_specs=pl.BlockSpec((TPAD, H), lambda e, po, nt: (0, 0)),
    )
    return pl.pallas_call(
        _ffn_body,
        grid_spec=grid_spec,
        out_shape=jax.ShapeDtypeStruct((TPAD, H), jnp.float32),
    )(po, nt, x_sorted, w1, b1.reshape(E, 1, F),
      w2, b2.reshape(E, 1, H))


def kernel(x, router_w, router_b, w1, b1, w2, b2):
    B, S, HH = x.shape
    xf = x.reshape(S, HH)
    pos2d, po2d, nt2d = _router(xf, router_w, router_b)
    pos = pos2d.reshape(T)
    sc_scatter, sc_gather = _sc_dispatch_kernels()
    x_sorted = sc_scatter(pos, xf)
    ffn = _ffn(po2d.reshape(E), nt2d.reshape(E), x_sorted, w1, b1, w2, b2)
    out = sc_gather(pos, ffn)
    return out.reshape(B, S, HH)
